# Initial kernel scaffold; baseline (speedup 1.0000x reference)
import jax
import jax.numpy as jnp
from jax.experimental import pallas as pl


def kernel(z, in_w, in_b, codebook, out_w, out_b):
    raise NotImplementedError("write your pallas kernel here")


# speedup vs baseline: 1.3472x; 1.3472x over previous
import jax
import jax.numpy as jnp
from jax.experimental import pallas as pl

B, SEQ, C, Hd = 8, 4096, 128, 16
OVERLAP, NUM_VQS, CB_DIM, CB_SIZE = 4, 6, 8, 1024
FIX = Hd * C
D = FIX * OVERLAP
Wd = SEQ // Hd
T = Wd // OVERLAP
N = B * T

CK = 2048
NCK = D // CK

_BF = jnp.bfloat16
_F32 = jnp.float32


def _vq_kernel(zp_ref, iw_ref, ib_ref, cbn_ref, cbnt_ref, ow_ref, ob_ref,
               res_ref, cm_ref):
    for kc in range(NCK):
        sl = pl.ds(kc * CK, CK)
        res_ref[:, sl] = zp_ref[:, sl]

    iota = jax.lax.broadcasted_iota(jnp.int32, (N, CB_SIZE), 1)
    row_b = jax.lax.broadcasted_iota(jnp.int32, (B, N), 1) // T
    bid = jax.lax.broadcasted_iota(jnp.int32, (B, N), 0)
    sel = (row_b == bid).astype(_F32)

    cm_acc = jnp.zeros((B, CB_DIM), _F32)
    for i in range(NUM_VQS):
        acc = jnp.zeros((N, CB_DIM), _F32)
        for kc in range(NCK):
            sl = pl.ds(kc * CK, CK)
            acc = acc + jax.lax.dot_general(
                res_ref[:, sl].astype(_BF), iw_ref[i, sl, :],
                (((1,), (0,)), ((), ())), preferred_element_type=_F32)
        ze = acc + ib_ref[i]
        nrm = jnp.sqrt(jnp.sum(ze * ze, axis=-1, keepdims=True))
        ze_n = ze / (nrm + 1e-8)
        sim = jax.lax.dot_general(
            ze_n.astype(_BF), cbnt_ref[i],
            (((1,), (0,)), ((), ())), preferred_element_type=_F32)
        m = jnp.max(sim, axis=-1, keepdims=True)
        idx = jnp.min(jnp.where(sim == m, iota, CB_SIZE), axis=-1,
                      keepdims=True)
        onehot = (iota == idx).astype(_F32)
        q = jax.lax.dot_general(
            onehot, cbn_ref[i], (((1,), (0,)), ((), ())),
            preferred_element_type=_F32,
            precision=jax.lax.Precision.HIGHEST)
        dq = ze_n - q
        cm_acc = cm_acc + jax.lax.dot_general(
            sel, dq * dq, (((1,), (0,)), ((), ())),
            preferred_element_type=_F32,
            precision=jax.lax.Precision.HIGHEST)
        qb = q.astype(_BF)
        for kc in range(NCK):
            sl = pl.ds(kc * CK, CK)
            zq_c = jax.lax.dot_general(
                qb, ow_ref[i, :, sl],
                (((1,), (0,)), ((), ())), preferred_element_type=_F32)
            zq_c = zq_c + ob_ref[i, :, sl]
            res_ref[:, sl] = res_ref[:, sl] - zq_c
    for kc in range(NCK):
        sl = pl.ds(kc * CK, CK)
        res_ref[:, sl] = zp_ref[:, sl] - res_ref[:, sl]
    cm = jnp.sum(cm_acc, axis=-1, keepdims=True) * (1.0 / (T * CB_DIM))
    cm_ref[...] = cm


def kernel(z, in_w, in_b, codebook, out_w, out_b):
    zz = z.reshape(B, Hd, Wd, C)
    zz = jnp.transpose(zz, (0, 2, 3, 1)).reshape(B, Wd, C * Hd)
    zp = zz.reshape(N, D)
    nrm = jnp.linalg.norm(codebook, axis=-1, keepdims=True)
    cb_n = codebook / (nrm + 1e-8)
    cbnt_bf = jnp.transpose(cb_n, (0, 2, 1)).astype(_BF)
    iw_bf = in_w.astype(_BF)
    ow_bf = out_w.astype(_BF)
    ib2 = in_b.reshape(NUM_VQS, 1, CB_DIM)
    ob2 = out_b.reshape(NUM_VQS, 1, D)

    out, cm2 = pl.pallas_call(
        _vq_kernel,
        out_shape=(
            jax.ShapeDtypeStruct((N, D), _F32),
            jax.ShapeDtypeStruct((B, 1), _F32),
        ),
    )(zp, iw_bf, ib2, cb_n, cbnt_bf, ow_bf, ob2)

    cm_loss = cm2.reshape(B)
    zq = out.reshape(B, Wd, C, Hd)
    zq_out = jnp.transpose(zq, (0, 3, 1, 2)).reshape(B, SEQ, C)
    return zq_out, cm_loss, cm_loss


# speedup vs baseline: 1.4066x; 1.0441x over previous
import jax
import jax.numpy as jnp
from jax.experimental import pallas as pl
from jax.experimental.pallas import tpu as pltpu

B, SEQ, C, Hd = 8, 4096, 128, 16
OVERLAP, NUM_VQS, CB_DIM, CB_SIZE = 4, 6, 8, 1024
FIX = Hd * C
D = FIX * OVERLAP
Wd = SEQ // Hd
T = Wd // OVERLAP
N = B * T

CK = 2048
NCK = D // CK

_BF = jnp.bfloat16
_F32 = jnp.float32


def _vq_kernel(zp_ref, iw_ref, ib_ref, cbn_bf_ref, cbnt_ref, ow_ref, ob_ref,
               res_ref, cm_ref, rbf_ref):
    for kc in range(NCK):
        sl = pl.ds(kc * CK, CK)
        zc = zp_ref[:, sl]
        res_ref[:, sl] = zc
        rbf_ref[:, sl] = zc.astype(_BF)

    iota = jax.lax.broadcasted_iota(jnp.int32, (N, CB_SIZE), 1)
    row_b = jax.lax.broadcasted_iota(jnp.int32, (B, N), 1) // T
    bid = jax.lax.broadcasted_iota(jnp.int32, (B, N), 0)
    sel = (row_b == bid).astype(_BF)

    cm_acc = jnp.zeros((B, CB_DIM), _F32)
    for i in range(NUM_VQS):
        ze = jax.lax.dot_general(
            rbf_ref[...], iw_ref[i],
            (((1,), (0,)), ((), ())), preferred_element_type=_F32)
        ze = ze + ib_ref[i]
        nrm = jnp.sqrt(jnp.sum(ze * ze, axis=-1, keepdims=True))
        ze_n = ze / (nrm + 1e-8)
        sim = jax.lax.dot_general(
            ze_n.astype(_BF), cbnt_ref[i],
            (((1,), (0,)), ((), ())), preferred_element_type=_F32)
        m = jnp.max(sim, axis=-1, keepdims=True)
        idx = jnp.min(jnp.where(sim == m, iota, CB_SIZE), axis=-1,
                      keepdims=True)
        onehot = (iota == idx).astype(_BF)
        qf = jax.lax.dot_general(
            onehot, cbn_bf_ref[i], (((1,), (0,)), ((), ())),
            preferred_element_type=_F32)
        qb = qf.astype(_BF)
        dq = ze_n - qf
        cm_acc = cm_acc + jax.lax.dot_general(
            sel, (dq * dq).astype(_BF), (((1,), (0,)), ((), ())),
            preferred_element_type=_F32)
        for kc in range(NCK):
            sl = pl.ds(kc * CK, CK)
            zq_c = jax.lax.dot_general(
                qb, ow_ref[i, :, sl],
                (((1,), (0,)), ((), ())), preferred_element_type=_F32)
            rc = res_ref[:, sl] - (zq_c + ob_ref[i, :, sl])
            res_ref[:, sl] = rc
            if i < NUM_VQS - 1:
                rbf_ref[:, sl] = rc.astype(_BF)
    for kc in range(NCK):
        sl = pl.ds(kc * CK, CK)
        res_ref[:, sl] = zp_ref[:, sl] - res_ref[:, sl]
    cm = jnp.sum(cm_acc, axis=-1, keepdims=True) * (1.0 / (T * CB_DIM))
    cm_ref[...] = cm


def kernel(z, in_w, in_b, codebook, out_w, out_b):
    zz = z.reshape(B, Hd, Wd, C)
    zz = jnp.transpose(zz, (0, 2, 3, 1)).reshape(B, Wd, C * Hd)
    zp = zz.reshape(N, D)
    nrm = jnp.linalg.norm(codebook, axis=-1, keepdims=True)
    cb_n = codebook / (nrm + 1e-8)
    cbn_bf = cb_n.astype(_BF)
    cbnt_bf = jnp.transpose(cb_n, (0, 2, 1)).astype(_BF)
    iw_bf = in_w.astype(_BF)
    ow_bf = out_w.astype(_BF)
    ib2 = in_b.reshape(NUM_VQS, 1, CB_DIM)
    ob2 = out_b.reshape(NUM_VQS, 1, D)

    out, cm2 = pl.pallas_call(
        _vq_kernel,
        out_shape=(
            jax.ShapeDtypeStruct((N, D), _F32),
            jax.ShapeDtypeStruct((B, 1), _F32),
        ),
        scratch_shapes=[pltpu.VMEM((N, D), _BF)],
    )(zp, iw_bf, ib2, cbn_bf, cbnt_bf, ow_bf, ob2)

    cm_loss = cm2.reshape(B)
    zq = out.reshape(B, Wd, C, Hd)
    zq_out = jnp.transpose(zq, (0, 3, 1, 2)).reshape(B, SEQ, C)
    return zq_out, cm_loss, cm_loss
